# baseline (speedup 1.0000x reference)
import functools

import jax
import jax.numpy as jnp
from jax import lax
from jax.experimental import pallas as pl
from jax.experimental.pallas import tpu as pltpu
from jax.experimental.pallas import tpu_sc as plsc

N_CORES = 2
N_SUB = 16
LANES = 16
CHUNK = 128


def _cdiv(a, b):
    return (a + b - 1) // b


def _deg_body(n_nodes, kd, col_hbm, w_hbm, deg_hbm, col_v, w_v, zbuf, deg_sp):
    c = lax.axis_index("c")
    s = lax.axis_index("s")
    wid = c * N_SUB + s
    base = pl.multiple_of(wid * kd, 8)

    pltpu.sync_copy(col_hbm.at[pl.ds(base, kd)], col_v)
    pltpu.sync_copy(w_hbm.at[pl.ds(base, kd)], w_v)

    @pl.when(s == 0)
    def _zero():
        zv = jnp.zeros((LANES,), jnp.float32)

        def zb(i, _):
            zbuf[pl.ds(i * LANES, LANES)] = zv
            return 0

        lax.fori_loop(0, n_nodes // LANES, zb, 0)
        pltpu.sync_copy(zbuf, deg_sp)

    plsc.subcore_barrier()

    def body(j, _):
        pltpu.sync_copy(w_v.at[j], deg_sp.at[col_v.at[j]], add=True)
        return 0

    lax.fori_loop(0, kd, body, 0)
    plsc.subcore_barrier()

    @pl.when(s == 0)
    def _out():
        pltpu.sync_copy(deg_sp, deg_hbm.at[c, 0])


def _deg_call(col2d, w2d, n_nodes):
    rows = col2d.shape[0]
    kd = rows // (N_CORES * N_SUB)
    mesh = plsc.VectorSubcoreMesh(core_axis_name="c", subcore_axis_name="s")
    kern = pl.kernel(
        functools.partial(_deg_body, n_nodes, kd),
        out_type=jax.ShapeDtypeStruct((N_CORES, 1, n_nodes), jnp.float32),
        mesh=mesh,
        scratch_types=[
            pltpu.VMEM((kd, CHUNK), jnp.int32),
            pltpu.VMEM((kd, CHUNK), jnp.float32),
            pltpu.VMEM((n_nodes,), jnp.float32),
            pltpu.VMEM_SHARED((n_nodes,), jnp.float32),
        ],
    )
    return kern(col2d, w2d)


def _fuse_body(x_ref, w_ref, degp_ref, h2_ref, dis_ref):
    h = jnp.dot(x_ref[...], w_ref[...], preferred_element_type=jnp.float32)
    deg = 1.0 + degp_ref[0, 0, :] + degp_ref[0, 1, :]
    dis = lax.rsqrt(deg)
    dis_ref[0, 0, :] = dis
    h2_ref[...] = h * dis[:, None]


def _fuse_call(x, W, degp, blk):
    n, cin = x.shape
    hid = W.shape[1]
    g = n // blk
    degp3 = degp.reshape(N_CORES, g, blk).transpose(1, 0, 2)
    return pl.pallas_call(
        _fuse_body,
        grid=(g,),
        in_specs=[
            pl.BlockSpec((blk, cin), lambda i: (i, 0)),
            pl.BlockSpec((cin, hid), lambda i: (0, 0)),
            pl.BlockSpec((1, N_CORES, blk), lambda i: (i, 0, 0)),
        ],
        out_specs=[
            pl.BlockSpec((blk, hid), lambda i: (i, 0)),
            pl.BlockSpec((1, 1, blk), lambda i: (i, 0, 0)),
        ],
        out_shape=[
            jax.ShapeDtypeStruct((n, hid), jnp.float32),
            jax.ShapeDtypeStruct((g, 1, blk), jnp.float32),
        ],
    )(x, W, degp3)


def _mp_body(n_nodes, hid, kb, h2_hbm, row_hbm, col_hbm, w_hbm, acc_hbm,
             row_v, colb, wb, msgs, acc_sp, sem0, sem1):
    c = lax.axis_index("c")
    s = lax.axis_index("s")
    wid = c * N_SUB + s
    nps = (n_nodes // N_SUB) // 8 * 8
    rem = n_nodes - nps * N_SUB
    r0 = pl.multiple_of(s * nps, 8)

    base = pl.multiple_of(wid * kb, 8)
    pltpu.sync_copy(row_hbm.at[pl.ds(base, kb)], row_v)

    @pl.when(c == 0)
    def _init_h():
        pltpu.sync_copy(h2_hbm.at[pl.ds(r0, nps)], acc_sp.at[pl.ds(r0, nps)])
        if rem:
            @pl.when(s == 0)
            def _rem_h():
                rb = nps * N_SUB
                pltpu.sync_copy(h2_hbm.at[pl.ds(rb, rem)],
                                acc_sp.at[pl.ds(rb, rem)])

    @pl.when(c == 1)
    def _init_z():
        def zrow(r, _):
            for q in range(CHUNK // LANES):
                msgs[0, r, pl.ds(q * LANES, LANES)] = jnp.zeros(
                    (LANES,), jnp.float32)
            return 0

        lax.fori_loop(0, CHUNK, zrow, 0)
        nfull = nps // CHUNK
        ztail = nps - nfull * CHUNK
        for k in range(nfull):
            pltpu.sync_copy(msgs.at[0],
                            acc_sp.at[pl.ds(r0 + k * CHUNK, CHUNK)])
        if ztail:
            pltpu.sync_copy(msgs.at[0, pl.ds(0, ztail)],
                            acc_sp.at[pl.ds(r0 + nfull * CHUNK, ztail)])
        if rem:
            @pl.when(s == 0)
            def _rem_z():
                rb = nps * N_SUB
                pltpu.sync_copy(msgs.at[0, pl.ds(0, rem)],
                                acc_sp.at[pl.ds(rb, rem)])

    plsc.subcore_barrier()

    sems = (sem0, sem1)

    def fetch(j, buf):
        pltpu.async_copy(h2_hbm.at[row_v.at[j]], msgs.at[buf], sems[buf])
        pltpu.async_copy(col_hbm.at[base + j], colb.at[buf], sems[buf])
        pltpu.async_copy(w_hbm.at[base + j], wb.at[buf], sems[buf])

    def wait(buf):
        pltpu.make_async_copy(h2_hbm.at[pl.ds(0, CHUNK)], msgs.at[buf],
                              sems[buf]).wait()
        pltpu.make_async_copy(col_hbm.at[0], colb.at[buf], sems[buf]).wait()
        pltpu.make_async_copy(w_hbm.at[0], wb.at[buf], sems[buf]).wait()

    fetch(0, 0)
    fetch(1, 1)

    t_total = kb // 2

    def chunk(j, buf, issue_next):
        wait(buf)

        for g in range(CHUNK // LANES):
            wv = wb[buf, pl.ds(g * LANES, LANES)]
            for el in range(LANES):
                e = g * LANES + el
                we = wv[el]
                for q in range(hid // LANES):
                    sl = pl.ds(q * LANES, LANES)
                    msgs[buf, e, sl] = msgs[buf, e, sl] * we

        pltpu.sync_copy(msgs.at[buf], acc_sp.at[colb.at[buf]], add=True)

        @pl.when(issue_next)
        def _():
            fetch(j + 2, buf)

    def body(t, _):
        issue = t < t_total - 1
        chunk(2 * t, 0, issue)
        chunk(2 * t + 1, 1, issue)
        return 0

    lax.fori_loop(0, t_total, body, 0)
    plsc.subcore_barrier()

    pltpu.sync_copy(acc_sp.at[pl.ds(r0, nps)], acc_hbm.at[c, pl.ds(r0, nps)])
    if rem:
        @pl.when(s == 0)
        def _out_rem():
            rb = nps * N_SUB
            pltpu.sync_copy(acc_sp.at[pl.ds(rb, rem)],
                            acc_hbm.at[c, pl.ds(rb, rem)])


def _mp_call(h2, row2d, col2d, w2d, n_nodes):
    hid = h2.shape[1]
    rows = row2d.shape[0]
    kb = rows // (N_CORES * N_SUB)
    mesh = plsc.VectorSubcoreMesh(core_axis_name="c", subcore_axis_name="s")
    kern = pl.kernel(
        functools.partial(_mp_body, n_nodes, hid, kb),
        out_type=jax.ShapeDtypeStruct((N_CORES, n_nodes, hid), jnp.float32),
        mesh=mesh,
        scratch_types=[
            pltpu.VMEM((kb, CHUNK), jnp.int32),
            pltpu.VMEM((2, CHUNK), jnp.int32),
            pltpu.VMEM((2, CHUNK), jnp.float32),
            pltpu.VMEM((2, CHUNK, hid), jnp.float32),
            pltpu.VMEM_SHARED((n_nodes, hid), jnp.float32),
            pltpu.SemaphoreType.DMA,
            pltpu.SemaphoreType.DMA,
        ],
    )
    return kern(h2, row2d, col2d, w2d)


def _final_body(acc_ref, dis_ref, b_ref, a_ref, out_ref):
    acc = acc_ref[0] + acc_ref[1]
    o = acc * dis_ref[0, 0, :][:, None] + b_ref[...][None, :]
    out_ref[...] = jnp.maximum(o, 0.0) + a_ref[...][None, :] * jnp.minimum(o, 0.0)


def _final_call(acc2, dis, b, prelu_a, blk):
    n = acc2.shape[1]
    hid = acc2.shape[2]
    g = n // blk
    return pl.pallas_call(
        _final_body,
        grid=(g,),
        in_specs=[
            pl.BlockSpec((N_CORES, blk, hid), lambda i: (0, i, 0)),
            pl.BlockSpec((1, 1, blk), lambda i: (i, 0, 0)),
            pl.BlockSpec((hid,), lambda i: (0,)),
            pl.BlockSpec((hid,), lambda i: (0,)),
        ],
        out_specs=pl.BlockSpec((blk, hid), lambda i: (i, 0)),
        out_shape=jax.ShapeDtypeStruct((n, hid), jnp.float32),
    )(acc2, dis, b, prelu_a)


def kernel(x, edge_index, weight, W, b, prelu_a):
    n, cin = x.shape
    hid = W.shape[1]
    e = edge_index.shape[1]

    row = edge_index[0].astype(jnp.int32)
    col = edge_index[1].astype(jnp.int32)
    w = weight.astype(jnp.float32)

    unit = N_CORES * N_SUB * CHUNK * 8
    ep = _cdiv(e, unit) * unit
    pad = ep - e
    if pad:
        spread = jnp.arange(pad, dtype=jnp.int32) % jnp.int32(n)
        row = jnp.concatenate([row, spread])
        col = jnp.concatenate([col, spread])
        w = jnp.concatenate([w, jnp.zeros((pad,), jnp.float32)])
    row2d = row.reshape(ep // CHUNK, CHUNK)
    col2d = col.reshape(ep // CHUNK, CHUNK)
    w2d = w.reshape(ep // CHUNK, CHUNK)

    blk = 1000
    degp = _deg_call(col2d, w2d, n).reshape(N_CORES, n)
    h2, dis = _fuse_call(x, W, degp, blk)
    acc2 = _mp_call(h2, row2d, col2d, w2d, n)
    return _final_call(acc2, dis, b, prelu_a, blk)
